# Initial kernel scaffold; baseline (speedup 1.0000x reference)
import jax
import jax.numpy as jnp
from jax.experimental import pallas as pl


def kernel(node_features, edge_features, edge_sources, edge_targets, W_init, b_init, W_edge, b_edge, gru_kernel, gru_rkernel, gru_bias, Wi, bi, Wj, bj):
    raise NotImplementedError("write your pallas kernel here")


# speedup vs baseline: 1.2211x; 1.2211x over previous
import functools

import jax
import jax.numpy as jnp
from jax import lax
from jax.experimental import pallas as pl
from jax.experimental.pallas import tpu as pltpu
from jax.experimental.pallas import tpu_sc as plsc

NC = 2
NS = 16
NW = NC * NS
CH = 128


def _ceil_to(x, m):
    return (x + m - 1) // m * m


def _make_sc_gather(Npad, Ep, HP):
    EPW = Ep // NW
    NCH = EPW // CH
    mesh = plsc.VectorSubcoreMesh(core_axis_name="c", subcore_axis_name="s",
                                  num_cores=NC, num_subcores=NS)

    @functools.partial(
        pl.kernel, mesh=mesh,
        compiler_params=pltpu.CompilerParams(use_tc_tiling_on_sc=False),
        out_type=jax.ShapeDtypeStruct((Ep, HP), jnp.float32),
        scratch_types=[
            [pltpu.VMEM((CH,), jnp.int32) for _ in range(NCH)],
            pltpu.VMEM((EPW, HP), jnp.float32),
            pltpu.SemaphoreType.DMA,
            pltpu.SemaphoreType.DMA,
        ],
    )
    def gather_k(table_hbm, idx_hbm, out_hbm, idx_vs, rows_v, semi, semg):
        wid = lax.axis_index("c") * NS + lax.axis_index("s")
        base = wid * EPW
        idescs = [pltpu.async_copy(idx_hbm.at[pl.ds(base + b * CH, CH)],
                                   idx_vs[b], semi) for b in range(NCH)]
        for dsc in idescs:
            dsc.wait()
        grp = 20
        for g in range(0, NCH, grp):
            descs = []
            for b in range(g, min(g + grp, NCH)):
                descs.append(pltpu.async_copy(
                    table_hbm.at[idx_vs[b]],
                    rows_v.at[pl.ds(b * CH, CH)], semg))
            for dsc in descs:
                dsc.wait()
        pltpu.sync_copy(rows_v, out_hbm.at[pl.ds(base, EPW)])

    return gather_k


def _make_sc_scatter(Npad, Ep, MP):
    EPW = Ep // NW
    NCH = EPW // CH
    G = 8
    ZR = Npad // NS
    mesh = plsc.VectorSubcoreMesh(core_axis_name="c", subcore_axis_name="s",
                                  num_cores=NC, num_subcores=NS)

    @functools.partial(
        pl.kernel, mesh=mesh,
        compiler_params=pltpu.CompilerParams(use_tc_tiling_on_sc=False),
        out_type=jax.ShapeDtypeStruct((2 * Npad, MP), jnp.float32),
        scratch_types=[
            [pltpu.VMEM((CH,), jnp.int32) for _ in range(NCH)],
            pltpu.VMEM((G, CH, MP), jnp.float32),
            pltpu.VMEM_SHARED((Npad, MP), jnp.float32),
            pltpu.SemaphoreType.DMA,
        ],
    )
    def scatter_k(rows_hbm, tgt_hbm, zeros_hbm, out_hbm, idx_vs, rows_v, acc,
                  sem):
        c = lax.axis_index("c")
        s = lax.axis_index("s")
        wid = c * NS + s
        base = wid * EPW
        idescs = [pltpu.async_copy(tgt_hbm.at[pl.ds(base + b * CH, CH)],
                                   idx_vs[b], sem) for b in range(NCH)]
        pltpu.sync_copy(zeros_hbm.at[pl.ds(s * ZR, ZR)],
                        acc.at[pl.ds(s * ZR, ZR)])
        for dsc in idescs:
            dsc.wait()
        plsc.subcore_barrier()
        for g in range(NCH // G):
            pltpu.sync_copy(rows_hbm.at[pl.ds(wid * NCH + g * G, G)], rows_v)
            for j in range(G):
                pltpu.sync_copy(rows_v.at[j], acc.at[idx_vs[g * G + j]],
                                add=True)
        plsc.subcore_barrier()
        pltpu.sync_copy(acc.at[pl.ds(s * ZR, ZR)],
                        out_hbm.at[pl.ds(c * Npad + s * ZR, ZR)])

    return scatter_k


def _prep_body(nf_ref, w_ref, b_ref, h_ref, ht_ref, *, N, BN, HP):
    i = pl.program_id(0)
    h = jnp.dot(nf_ref[...], w_ref[...],
                preferred_element_type=jnp.float32) + b_ref[...]
    rows = lax.broadcasted_iota(jnp.int32, (BN, HP), 0) + i * BN
    h = jnp.where(rows < N, h, 0.0)
    h_ref[...] = h
    ht_ref[...] = h.T


def _msgs_body(neigh_ref, ef_ref, w_ref, o_ref, *, DE, M, BE, MP):
    T = jnp.dot(neigh_ref[...], w_ref[...],
                preferred_element_type=jnp.float32)
    ef = ef_ref[...]
    acc = T[:, DE * M:DE * M + M]
    for d in range(DE):
        acc = acc + ef[:, d:d + 1] * T[:, d * M:(d + 1) * M]
    o_ref[...] = jnp.concatenate(
        [acc, jnp.zeros((BE, MP - M), jnp.float32)], axis=1)


def _gru_body(ht_ref, p0_ref, p1_ref, rkt_ref, kcol_ref, bin_ref, brec_ref,
              hto_ref, ho_ref, *, N, NB, H, M, HP):
    i = pl.program_id(0)
    h0 = ht_ref[...]
    h10 = h0[0:H, :]
    mT = (p0_ref[...] + p1_ref[...]).T
    kcol = kcol_ref[...]
    b_in = bin_ref[...]
    b_rec = brec_ref[...]
    rkt = rkt_ref[...]
    h = jnp.zeros((H, NB), jnp.float32)
    for t in range(H + M):
        x = h10[t:t + 1, :] if t < H else mT[t - H:t - H + 1, :]
        xkb = kcol * x + b_in
        A = jnp.dot(rkt, h, preferred_element_type=jnp.float32) + b_rec
        z = jax.nn.sigmoid(xkb[0:H] + A[0:H])
        r = jax.nn.sigmoid(xkb[H:2 * H] + A[H:2 * H])
        hh = jnp.tanh(xkb[2 * H:3 * H] + r * A[2 * H:3 * H])
        h = z * h + (1.0 - z) * hh
    lanes = lax.broadcasted_iota(jnp.int32, (H, NB), 1) + i * NB
    h = jnp.where(lanes < N, h, 0.0)
    ht_new = jnp.concatenate([h, jnp.zeros((HP - H, NB), jnp.float32)], axis=0)
    hto_ref[...] = ht_new
    ho_ref[...] = ht_new.T


def _readout_body(ht_ref, h0t_ref, wia_ref, wib_ref, wj_ref, bi_ref, bj_ref,
                  o_ref, *, N, NB, H):
    i = pl.program_id(0)
    h = ht_ref[0:H, :]
    h0 = h0t_ref[0:H, :]
    iv = (jnp.dot(wia_ref[...], h, preferred_element_type=jnp.float32)
          + jnp.dot(wib_ref[...], h0, preferred_element_type=jnp.float32)
          + bi_ref[0, 0])
    jv = jnp.dot(wj_ref[...], h, preferred_element_type=jnp.float32) + bj_ref[0, 0]
    lanes = lax.broadcasted_iota(jnp.int32, (1, NB), 1) + i * NB
    s = jnp.sum(jnp.where(lanes < N, iv * jv, 0.0)).reshape(1, 1)

    @pl.when(i == 0)
    def _():
        o_ref[...] = jnp.zeros((1, 1), jnp.float32)
    o_ref[...] += s


def kernel(node_features, edge_features, edge_sources, edge_targets,
           W_init, b_init, W_edge, b_edge,
           gru_kernel, gru_rkernel, gru_bias,
           Wi, bi, Wj, bj):
    f32 = jnp.float32
    N, DF = node_features.shape
    E, DE = edge_features.shape
    H = W_init.shape[1]
    M = W_edge.shape[1] // H
    ITERS = 3
    HP = 16
    MP = 32
    BN = 2048
    BE = 2048
    Npad = _ceil_to(N, BN)
    Ep = _ceil_to(E, NW * CH)
    EPW = Ep // NW
    NCH = EPW // CH

    nf_pad = jnp.concatenate(
        [node_features, jnp.zeros((Npad - N, DF), f32)], axis=0)
    ef_pad = jnp.concatenate(
        [edge_features, jnp.zeros((Ep - E, DE), f32)], axis=0)
    src_pad = jnp.concatenate(
        [edge_sources, jnp.full((Ep - E,), N, jnp.int32)])
    tgt_pad = jnp.concatenate(
        [edge_targets, jnp.full((Ep - E,), N, jnp.int32)])

    W16 = jnp.concatenate([W_init, jnp.zeros((DF, HP - H), f32)], axis=1)
    b16 = jnp.concatenate([b_init, jnp.zeros((HP - H,), f32)]).reshape(1, HP)
    Wall = W_edge.reshape(DE, M, H).transpose(2, 0, 1).reshape(H, DE * M)
    Wall2 = jnp.concatenate([Wall, b_edge.reshape(M, H).T], axis=1)
    Wall2p = jnp.concatenate(
        [Wall2, jnp.zeros((HP - H, DE * M + M), f32)], axis=0)
    rkt = gru_rkernel.T
    kcol = gru_kernel.reshape(3 * H, 1)
    b_in = gru_bias[0].reshape(3 * H, 1)
    b_rec = gru_bias[1].reshape(3 * H, 1)
    wia = Wi[:H, 0].reshape(1, H)
    wib = Wi[H:, 0].reshape(1, H)
    wjr = Wj[:, 0].reshape(1, H)
    bi2 = bi.reshape(1, 1)
    bj2 = bj.reshape(1, 1)
    zeros_acc = jnp.zeros((Npad, MP), f32)

    nblk = Npad // BN
    hid, h0t = pl.pallas_call(
        functools.partial(_prep_body, N=N, BN=BN, HP=HP),
        grid=(nblk,),
        in_specs=[
            pl.BlockSpec((BN, DF), lambda i: (i, 0)),
            pl.BlockSpec((DF, HP), lambda i: (0, 0)),
            pl.BlockSpec((1, HP), lambda i: (0, 0)),
        ],
        out_specs=[
            pl.BlockSpec((BN, HP), lambda i: (i, 0)),
            pl.BlockSpec((HP, BN), lambda i: (0, i)),
        ],
        out_shape=[
            jax.ShapeDtypeStruct((Npad, HP), f32),
            jax.ShapeDtypeStruct((HP, Npad), f32),
        ],
    )(nf_pad, W16, b16)

    gather_k = _make_sc_gather(Npad, Ep, HP)
    scatter_k = _make_sc_scatter(Npad, Ep, MP)

    msgs_call = pl.pallas_call(
        functools.partial(_msgs_body, DE=DE, M=M, BE=BE, MP=MP),
        grid=(Ep // BE,),
        in_specs=[
            pl.BlockSpec((BE, HP), lambda i: (i, 0)),
            pl.BlockSpec((BE, DE), lambda i: (i, 0)),
            pl.BlockSpec((HP, DE * M + M), lambda i: (0, 0)),
        ],
        out_specs=pl.BlockSpec((BE, MP), lambda i: (i, 0)),
        out_shape=jax.ShapeDtypeStruct((Ep, MP), f32),
    )

    nb2 = Npad // BN
    gru_call = pl.pallas_call(
        functools.partial(_gru_body, N=N, NB=BN, H=H, M=M, HP=HP),
        grid=(nb2,),
        in_specs=[
            pl.BlockSpec((HP, BN), lambda i: (0, i)),
            pl.BlockSpec((BN, MP), lambda i: (i, 0)),
            pl.BlockSpec((BN, MP), lambda i, _n=nb2: (i + _n, 0)),
            pl.BlockSpec((3 * H, H), lambda i: (0, 0)),
            pl.BlockSpec((3 * H, 1), lambda i: (0, 0)),
            pl.BlockSpec((3 * H, 1), lambda i: (0, 0)),
            pl.BlockSpec((3 * H, 1), lambda i: (0, 0)),
        ],
        out_specs=[
            pl.BlockSpec((HP, BN), lambda i: (0, i)),
            pl.BlockSpec((BN, HP), lambda i: (i, 0)),
        ],
        out_shape=[
            jax.ShapeDtypeStruct((HP, Npad), f32),
            jax.ShapeDtypeStruct((Npad, HP), f32),
        ],
    )

    ht = h0t
    for _ in range(ITERS):
        neigh = gather_k(hid, src_pad)
        msgs = msgs_call(neigh, ef_pad, Wall2p)
        parts = scatter_k(msgs.reshape(Ep // CH, CH, MP), tgt_pad, zeros_acc)
        ht, hid = gru_call(ht, parts, parts, rkt, kcol, b_in, b_rec)

    out = pl.pallas_call(
        functools.partial(_readout_body, N=N, NB=BN, H=H),
        grid=(nb2,),
        in_specs=[
            pl.BlockSpec((HP, BN), lambda i: (0, i)),
            pl.BlockSpec((HP, BN), lambda i: (0, i)),
            pl.BlockSpec((1, H), lambda i: (0, 0)),
            pl.BlockSpec((1, H), lambda i: (0, 0)),
            pl.BlockSpec((1, H), lambda i: (0, 0)),
            pl.BlockSpec((1, 1), lambda i: (0, 0)),
            pl.BlockSpec((1, 1), lambda i: (0, 0)),
        ],
        out_specs=pl.BlockSpec((1, 1), lambda i: (0, 0)),
        out_shape=jax.ShapeDtypeStruct((1, 1), f32),
    )(ht, h0t, wia, wib, wjr, bi2, bj2)
    return out.reshape(1)


# speedup vs baseline: 3.5642x; 2.9189x over previous
import functools

import jax
import jax.numpy as jnp
from jax import lax
from jax.experimental import pallas as pl
from jax.experimental.pallas import tpu as pltpu
from jax.experimental.pallas import tpu_sc as plsc

NC = 2
NS = 16
NW = NC * NS
CH = 128


def _ceil_to(x, m):
    return (x + m - 1) // m * m


def _make_sc_gather(Npad, Ep, HP):
    EPW = Ep // NW
    NCH = EPW // CH
    mesh = plsc.VectorSubcoreMesh(core_axis_name="c", subcore_axis_name="s",
                                  num_cores=NC, num_subcores=NS)

    @functools.partial(
        pl.kernel, mesh=mesh,
        compiler_params=pltpu.CompilerParams(use_tc_tiling_on_sc=False),
        out_type=jax.ShapeDtypeStruct((Ep, HP), jnp.float32),
        scratch_types=[
            [pltpu.VMEM((CH,), jnp.int32) for _ in range(NCH)],
            pltpu.VMEM((EPW, HP), jnp.float32),
            pltpu.SemaphoreType.DMA,
            pltpu.SemaphoreType.DMA,
        ],
    )
    def gather_k(table_hbm, idx_hbm, out_hbm, idx_vs, rows_v, semi, semg):
        wid = lax.axis_index("c") * NS + lax.axis_index("s")
        base = wid * EPW
        idescs = [pltpu.async_copy(idx_hbm.at[pl.ds(base + b * CH, CH)],
                                   idx_vs[b], semi) for b in range(NCH)]
        for dsc in idescs:
            dsc.wait()
        grp = 20
        for g in range(0, NCH, grp):
            descs = []
            for b in range(g, min(g + grp, NCH)):
                descs.append(pltpu.async_copy(
                    table_hbm.at[idx_vs[b]],
                    rows_v.at[pl.ds(b * CH, CH)], semg))
            for dsc in descs:
                dsc.wait()
        pltpu.sync_copy(rows_v, out_hbm.at[pl.ds(base, EPW)])

    return gather_k


def _make_sc_scatter(Npad, Ep, MP):
    EPW = Ep // NW
    NCH = EPW // CH
    G = 8
    ZR = Npad // NS
    mesh = plsc.VectorSubcoreMesh(core_axis_name="c", subcore_axis_name="s",
                                  num_cores=NC, num_subcores=NS)

    @functools.partial(
        pl.kernel, mesh=mesh,
        compiler_params=pltpu.CompilerParams(use_tc_tiling_on_sc=False),
        out_type=jax.ShapeDtypeStruct((2 * Npad, MP), jnp.float32),
        scratch_types=[
            [pltpu.VMEM((CH,), jnp.int32) for _ in range(NCH)],
            pltpu.VMEM((G * CH, MP), jnp.float32),
            pltpu.VMEM_SHARED((Npad, MP), jnp.float32),
            pltpu.SemaphoreType.DMA,
        ],
    )
    def scatter_k(rows_hbm, tgt_hbm, zeros_hbm, out_hbm, idx_vs, rows_v, acc,
                  sem):
        c = lax.axis_index("c")
        s = lax.axis_index("s")
        wid = c * NS + s
        base = wid * EPW
        idescs = [pltpu.async_copy(tgt_hbm.at[pl.ds(base + b * CH, CH)],
                                   idx_vs[b], sem) for b in range(NCH)]
        pltpu.sync_copy(zeros_hbm.at[pl.ds(s * ZR, ZR)],
                        acc.at[pl.ds(s * ZR, ZR)])
        for dsc in idescs:
            dsc.wait()
        plsc.subcore_barrier()
        for g in range(NCH // G):
            pltpu.sync_copy(rows_hbm.at[pl.ds(base + g * G * CH, G * CH)],
                            rows_v)
            for j in range(G):
                pltpu.sync_copy(rows_v.at[pl.ds(j * CH, CH)],
                                acc.at[idx_vs[g * G + j]], add=True)
        plsc.subcore_barrier()
        pltpu.sync_copy(acc.at[pl.ds(s * ZR, ZR)],
                        out_hbm.at[pl.ds(c * Npad + s * ZR, ZR)])

    return scatter_k


def _prep_body(nf_ref, w_ref, b_ref, h_ref, ht_ref, *, N, BN, HP):
    i = pl.program_id(0)
    h = jnp.dot(nf_ref[...], w_ref[...],
                preferred_element_type=jnp.float32) + b_ref[...]
    rows = lax.broadcasted_iota(jnp.int32, (BN, HP), 0) + i * BN
    h = jnp.where(rows < N, h, 0.0)
    h_ref[...] = h
    ht_ref[...] = h.T


def _msgs_body(neigh_ref, ef_ref, wt_ref, o_ref, z_ref, *, DE, H, BE, HP):
    nT = neigh_ref[...].T
    efT = ef_ref[...].T
    for h in range(H):
        z_ref[h * DE:(h + 1) * DE, :] = efT * nT[h:h + 1, :]
    z_ref[H * DE:H * DE + HP, :] = nT
    msgsT = jnp.dot(wt_ref[...], z_ref[...],
                    preferred_element_type=jnp.float32)
    o_ref[...] = msgsT.T


def _gru_body(ht_ref, p0_ref, p1_ref, rkt_ref, kcol_ref, bin_ref, brec_ref,
              hto_ref, ho_ref, *, N, NB, H, M, HP):
    i = pl.program_id(0)
    h0 = ht_ref[...]
    h10 = h0[0:H, :]
    mT = (p0_ref[...] + p1_ref[...]).T
    kcol = kcol_ref[...]
    b_in = bin_ref[...]
    b_rec = brec_ref[...]
    rkt = rkt_ref[...]
    h = jnp.zeros((H, NB), jnp.float32)
    for t in range(H + M):
        x = h10[t:t + 1, :] if t < H else mT[t - H:t - H + 1, :]
        xkb = kcol * x + b_in
        A = jnp.dot(rkt, h, preferred_element_type=jnp.float32) + b_rec
        z = jax.nn.sigmoid(xkb[0:H] + A[0:H])
        r = jax.nn.sigmoid(xkb[H:2 * H] + A[H:2 * H])
        hh = jnp.tanh(xkb[2 * H:3 * H] + r * A[2 * H:3 * H])
        h = z * h + (1.0 - z) * hh
    lanes = lax.broadcasted_iota(jnp.int32, (H, NB), 1) + i * NB
    h = jnp.where(lanes < N, h, 0.0)
    ht_new = jnp.concatenate([h, jnp.zeros((HP - H, NB), jnp.float32)], axis=0)
    hto_ref[...] = ht_new
    ho_ref[...] = ht_new.T


def _readout_body(ht_ref, h0t_ref, wia_ref, wib_ref, wj_ref, bi_ref, bj_ref,
                  o_ref, *, N, NB, H):
    i = pl.program_id(0)
    h = ht_ref[0:H, :]
    h0 = h0t_ref[0:H, :]
    iv = (jnp.dot(wia_ref[...], h, preferred_element_type=jnp.float32)
          + jnp.dot(wib_ref[...], h0, preferred_element_type=jnp.float32)
          + bi_ref[0, 0])
    jv = jnp.dot(wj_ref[...], h, preferred_element_type=jnp.float32) + bj_ref[0, 0]
    lanes = lax.broadcasted_iota(jnp.int32, (1, NB), 1) + i * NB
    s = jnp.sum(jnp.where(lanes < N, iv * jv, 0.0)).reshape(1, 1)

    @pl.when(i == 0)
    def _():
        o_ref[...] = jnp.zeros((1, 1), jnp.float32)
    o_ref[...] += s


def kernel(node_features, edge_features, edge_sources, edge_targets,
           W_init, b_init, W_edge, b_edge,
           gru_kernel, gru_rkernel, gru_bias,
           Wi, bi, Wj, bj):
    f32 = jnp.float32
    N, DF = node_features.shape
    E, DE = edge_features.shape
    H = W_init.shape[1]
    M = W_edge.shape[1] // H
    ITERS = 3
    HP = 16
    MP = 32
    BN = 2048
    BE = 2048
    Npad = _ceil_to(N, BN)
    Ep = _ceil_to(E, NW * CH)
    EPW = Ep // NW
    NCH = EPW // CH

    nf_pad = jnp.concatenate(
        [node_features, jnp.zeros((Npad - N, DF), f32)], axis=0)
    ef_pad = jnp.concatenate(
        [edge_features, jnp.zeros((Ep - E, DE), f32)], axis=0)
    src_pad = jnp.concatenate(
        [edge_sources, jnp.full((Ep - E,), N, jnp.int32)])
    tgt_pad = jnp.concatenate(
        [edge_targets, jnp.full((Ep - E,), N, jnp.int32)])

    W16 = jnp.concatenate([W_init, jnp.zeros((DF, HP - H), f32)], axis=1)
    b16 = jnp.concatenate([b_init, jnp.zeros((HP - H,), f32)]).reshape(1, HP)
    ZD = H * DE + HP
    Wt = jnp.concatenate(
        [W_edge.reshape(DE, M, H).transpose(1, 2, 0).reshape(M, H * DE),
         b_edge.reshape(M, H), jnp.zeros((M, HP - H), f32)], axis=1)
    Wt = jnp.concatenate([Wt, jnp.zeros((MP - M, ZD), f32)], axis=0)
    rkt = gru_rkernel.T
    kcol = gru_kernel.reshape(3 * H, 1)
    b_in = gru_bias[0].reshape(3 * H, 1)
    b_rec = gru_bias[1].reshape(3 * H, 1)
    wia = Wi[:H, 0].reshape(1, H)
    wib = Wi[H:, 0].reshape(1, H)
    wjr = Wj[:, 0].reshape(1, H)
    bi2 = bi.reshape(1, 1)
    bj2 = bj.reshape(1, 1)
    zeros_acc = jnp.zeros((Npad, MP), f32)

    nblk = Npad // BN
    hid, h0t = pl.pallas_call(
        functools.partial(_prep_body, N=N, BN=BN, HP=HP),
        grid=(nblk,),
        in_specs=[
            pl.BlockSpec((BN, DF), lambda i: (i, 0)),
            pl.BlockSpec((DF, HP), lambda i: (0, 0)),
            pl.BlockSpec((1, HP), lambda i: (0, 0)),
        ],
        out_specs=[
            pl.BlockSpec((BN, HP), lambda i: (i, 0)),
            pl.BlockSpec((HP, BN), lambda i: (0, i)),
        ],
        out_shape=[
            jax.ShapeDtypeStruct((Npad, HP), f32),
            jax.ShapeDtypeStruct((HP, Npad), f32),
        ],
    )(nf_pad, W16, b16)

    gather_k = _make_sc_gather(Npad, Ep, HP)
    scatter_k = _make_sc_scatter(Npad, Ep, MP)

    msgs_call = pl.pallas_call(
        functools.partial(_msgs_body, DE=DE, H=H, BE=BE, HP=HP),
        grid=(Ep // BE,),
        in_specs=[
            pl.BlockSpec((BE, HP), lambda i: (i, 0)),
            pl.BlockSpec((BE, DE), lambda i: (i, 0)),
            pl.BlockSpec((MP, ZD), lambda i: (0, 0)),
        ],
        out_specs=pl.BlockSpec((BE, MP), lambda i: (i, 0)),
        out_shape=jax.ShapeDtypeStruct((Ep, MP), f32),
        scratch_shapes=[pltpu.VMEM((ZD, BE), f32)],
    )

    nb2 = Npad // BN
    gru_call = pl.pallas_call(
        functools.partial(_gru_body, N=N, NB=BN, H=H, M=M, HP=HP),
        grid=(nb2,),
        in_specs=[
            pl.BlockSpec((HP, BN), lambda i: (0, i)),
            pl.BlockSpec((BN, MP), lambda i: (i, 0)),
            pl.BlockSpec((BN, MP), lambda i, _n=nb2: (i + _n, 0)),
            pl.BlockSpec((3 * H, H), lambda i: (0, 0)),
            pl.BlockSpec((3 * H, 1), lambda i: (0, 0)),
            pl.BlockSpec((3 * H, 1), lambda i: (0, 0)),
            pl.BlockSpec((3 * H, 1), lambda i: (0, 0)),
        ],
        out_specs=[
            pl.BlockSpec((HP, BN), lambda i: (0, i)),
            pl.BlockSpec((BN, HP), lambda i: (i, 0)),
        ],
        out_shape=[
            jax.ShapeDtypeStruct((HP, Npad), f32),
            jax.ShapeDtypeStruct((Npad, HP), f32),
        ],
    )

    ht = h0t
    for _ in range(ITERS):
        neigh = gather_k(hid, src_pad)
        msgs = msgs_call(neigh, ef_pad, Wt)
        parts = scatter_k(msgs, tgt_pad, zeros_acc)
        ht, hid = gru_call(ht, parts, parts, rkt, kcol, b_in, b_rec)

    out = pl.pallas_call(
        functools.partial(_readout_body, N=N, NB=BN, H=H),
        grid=(nb2,),
        in_specs=[
            pl.BlockSpec((HP, BN), lambda i: (0, i)),
            pl.BlockSpec((HP, BN), lambda i: (0, i)),
            pl.BlockSpec((1, H), lambda i: (0, 0)),
            pl.BlockSpec((1, H), lambda i: (0, 0)),
            pl.BlockSpec((1, H), lambda i: (0, 0)),
            pl.BlockSpec((1, 1), lambda i: (0, 0)),
            pl.BlockSpec((1, 1), lambda i: (0, 0)),
        ],
        out_specs=pl.BlockSpec((1, 1), lambda i: (0, 0)),
        out_shape=jax.ShapeDtypeStruct((1, 1), f32),
    )(ht, h0t, wia, wib, wjr, bi2, bj2)
    return out.reshape(1)
